# baseline (speedup 1.0000x reference)
import functools

import jax
import jax.numpy as jnp
import numpy as np
from jax import lax
from jax.experimental import pallas as pl
from jax.experimental.pallas import tpu as pltpu
from jax.experimental.pallas import tpu_sc as plsc

NC = 2
NS = 16
L = 16

R = 1280
NCHUNK = 8
B = 64


def _tc_project(x, W, Wp, A):
    N, IN = x.shape
    HC = W.shape[1]
    BLK = 2000 if N % 2000 == 0 else N

    def body(x_ref, w_ref, wp_ref, a_ref, xp_ref, aall_ref, xp16_ref):
        xv = x_ref[...]
        xp = jnp.dot(xv, w_ref[...], preferred_element_type=jnp.float32)
        xp_ref[...] = xp
        aall_ref[...] = jnp.dot(xp, a_ref[...], preferred_element_type=jnp.float32)
        xp16_ref[...] = jnp.dot(
            xv, wp_ref[...], preferred_element_type=jnp.float32
        ).astype(jnp.bfloat16)

    return pl.pallas_call(
        body,
        grid=(N // BLK,),
        in_specs=[
            pl.BlockSpec((BLK, IN), lambda i: (i, 0)),
            pl.BlockSpec((IN, HC), lambda i: (0, 0)),
            pl.BlockSpec((IN, HC), lambda i: (0, 0)),
            pl.BlockSpec((HC, 128), lambda i: (0, 0)),
        ],
        out_specs=[
            pl.BlockSpec((BLK, HC), lambda i: (i, 0)),
            pl.BlockSpec((BLK, 128), lambda i: (i, 0)),
            pl.BlockSpec((BLK, HC), lambda i: (i, 0)),
        ],
        out_shape=[
            jax.ShapeDtypeStruct((N, HC), jnp.float32),
            jax.ShapeDtypeStruct((N, 128), jnp.float32),
            jax.ShapeDtypeStruct((N, HC), jnp.bfloat16),
        ],
    )(x, W, Wp, A)


def _sc_weights(src, dst, a4, n_nodes):
    E_pad = src.shape[0]
    SPAN = E_pad // (NC * NS)
    N = n_nodes

    mesh = plsc.VectorSubcoreMesh(
        core_axis_name="c", subcore_axis_name="s", num_cores=NC, num_subcores=NS)

    @functools.partial(
        pl.kernel,
        mesh=mesh,
        compiler_params=pltpu.CompilerParams(
            use_tc_tiling_on_sc=False, needs_layout_passes=False),
        out_type=[
            jax.ShapeDtypeStruct((E_pad,), jnp.float32),
            jax.ShapeDtypeStruct((E_pad,), jnp.float32),
        ],
        scratch_types=[
            pltpu.VMEM((N * 4,), jnp.float32),
            pltpu.VMEM((SPAN,), jnp.int32),
            pltpu.VMEM((SPAN,), jnp.int32),
            pltpu.VMEM((SPAN,), jnp.float32),
            pltpu.VMEM((SPAN,), jnp.float32),
        ],
    )
    def k(src_hbm, dst_hbm, a4_hbm, w0_hbm, w1_hbm,
          a4_v, src_v, dst_v, w0_v, w1_v):
        cid = lax.axis_index("c")
        sid = lax.axis_index("s")
        base = (cid * NS + sid) * SPAN
        pltpu.sync_copy(a4_hbm, a4_v)
        pltpu.sync_copy(src_hbm.at[pl.ds(base, SPAN)], src_v)
        pltpu.sync_copy(dst_hbm.at[pl.ds(base, SPAN)], dst_v)

        def grp(g, _):
            s16 = src_v[pl.ds(g * L, L)]
            d16 = dst_v[pl.ds(g * L, L)]
            valid = d16 < jnp.int32(N)
            s4 = s16 * 4
            d4 = d16 * 4
            as0 = plsc.load_gather(a4_v, [s4], mask=valid)
            as1 = plsc.load_gather(a4_v, [s4 + 1], mask=valid)
            ad0 = plsc.load_gather(a4_v, [d4 + 2], mask=valid)
            ad1 = plsc.load_gather(a4_v, [d4 + 3], mask=valid)
            al0 = as0 + ad0
            al1 = as1 + ad1
            al0 = jnp.where(al0 > 0, al0, 0.2 * al0)
            al1 = jnp.where(al1 > 0, al1, 0.2 * al1)
            w0_v[pl.ds(g * L, L)] = jnp.where(valid, jnp.exp(al0), 0.0)
            w1_v[pl.ds(g * L, L)] = jnp.where(valid, jnp.exp(al1), 0.0)
            return 0

        lax.fori_loop(0, SPAN // L, grp, 0)
        pltpu.sync_copy(w0_v, w0_hbm.at[pl.ds(base, SPAN)])
        pltpu.sync_copy(w1_v, w1_hbm.at[pl.ds(base, SPAN)])

    return k(src, dst, a4)


def _sc_scatter(xp16, src, dst, w0, w1, bias):
    N, HC = xp16.shape
    KV = HC // L
    KW = HC // (2 * L)
    E_pad = src.shape[0]
    SPAN = E_pad // NS
    GROUPS = SPAN // L
    EBLK_G = 74
    EBLK = EBLK_G * L
    NBLK = GROUPS // EBLK_G
    NPAD = R * NCHUNK
    SH = R // NS

    assert GROUPS % EBLK_G == 0

    mesh = plsc.VectorSubcoreMesh(
        core_axis_name="c", subcore_axis_name="s", num_cores=NC, num_subcores=NS)

    buf_types = []
    for _ in range(2):
        buf_types += [
            pltpu.VMEM((B, HC), jnp.bfloat16),
            pltpu.VMEM((B,), jnp.int32),
            pltpu.VMEM((B,), jnp.int32),
            pltpu.VMEM((B,), jnp.float32),
            pltpu.VMEM((B,), jnp.float32),
            pltpu.SemaphoreType.DMA,
        ]

    @functools.partial(
        pl.kernel,
        mesh=mesh,
        compiler_params=pltpu.CompilerParams(
            use_tc_tiling_on_sc=False, needs_layout_passes=False),
        out_type=jax.ShapeDtypeStruct((NPAD, HC), jnp.float32),
        scratch_types=[
            pltpu.VMEM_SHARED((R, HC), jnp.float32),
            pltpu.VMEM_SHARED((NS, R), jnp.float32),
            pltpu.VMEM_SHARED((NS, R), jnp.float32),
            pltpu.VMEM((EBLK,), jnp.int32),
            pltpu.VMEM((EBLK,), jnp.int32),
            pltpu.VMEM((EBLK,), jnp.float32),
            pltpu.VMEM((EBLK,), jnp.float32),
            pltpu.VMEM((B, HC), jnp.float32),
            pltpu.VMEM((R,), jnp.float32),
            pltpu.VMEM((R,), jnp.float32),
            pltpu.VMEM((HC,), jnp.float32),
            pltpu.VMEM((NS, SH), jnp.float32),
            pltpu.VMEM((NS, SH), jnp.float32),
            pltpu.VMEM((L,), jnp.float32),
            pltpu.VMEM((L,), jnp.float32),
            pltpu.VMEM((B,), jnp.int32),
            pltpu.VMEM((B,), jnp.int32),
            pltpu.VMEM((B,), jnp.float32),
            pltpu.VMEM((B,), jnp.float32),
        ] + buf_types,
    )
    def k(xp16_hbm, src_hbm, dst_hbm, w0_hbm, w1_hbm, bias_hbm, out_hbm,
          slab, dall0, dall1, eb_src, eb_dst, eb_w0, eb_w1, rowsf,
          den0, den1, bias_v, dc0, dc1, inv0_v, inv1_v,
          st_src, st_loc, st_w0, st_w1,
          rows_a, wls_a, wll_a, ww0_a, ww1_a, gsem_a,
          rows_b, wls_b, wll_b, ww0_b, ww1_b, gsem_b):
        cid = lax.axis_index("c")
        sid = lax.axis_index("s")
        base = sid * SPAN
        pltpu.sync_copy(bias_hbm, bias_v)

        bufs = ((rows_a, wls_a, wll_a, ww0_a, ww1_a, gsem_a),
                (rows_b, wls_b, wll_b, ww0_b, ww1_b, gsem_b))

        zf = jnp.zeros((L,), jnp.float32)
        iot = lax.iota(jnp.int32, L)

        def resentinel():
            for b in range(B // L):
                st_src[pl.ds(b * L, L)] = (iot * 601 + b * 977) % N
                st_loc[pl.ds(b * L, L)] = (iot * 37 + b * 53) % R
                st_w0[pl.ds(b * L, L)] = zf
                st_w1[pl.ds(b * L, L)] = zf

        def snapshot(p):
            _, wls, wll, ww0, ww1, _ = bufs[p]
            for b in range(B // L):
                sl = pl.ds(b * L, L)
                wls[sl] = st_src[sl]
                wll[sl] = st_loc[sl]
                ww0[sl] = st_w0[sl]
                ww1[sl] = st_w1[sl]
            resentinel()

        def zero_rowsf():
            def zrow(e, _):
                row = rowsf.at[e]
                for kk in range(KV):
                    row[pl.ds(kk * L, L)] = zf
                return 0
            lax.fori_loop(0, B, zrow, 0)

        def start_gather(p):
            rows, wls = bufs[p][0], bufs[p][1]
            pltpu.make_async_copy(xp16_hbm.at[wls], rows, bufs[p][5]).start()

        def drain(p):
            rows, wls, wll, ww0, ww1, gsem = bufs[p]
            pltpu.make_async_copy(xp16_hbm.at[wls], rows, gsem).wait()

            def scale(e, _):
                ev = jnp.full((L,), e, jnp.int32)
                wv0 = plsc.load_gather(ww0, [ev])
                wv1 = plsc.load_gather(ww1, [ev])
                row16 = rows.at[e]
                rowf = rowsf.at[e]
                for kk in range(KW):
                    v32 = row16[pl.ds(kk * 2 * L, 2 * L)]
                    a, b = plsc.unpack(v32, format=plsc.PackFormat.INTERLEAVED)
                    w = wv0 if kk < KW // 2 else wv1
                    rowf[pl.ds(kk * 2 * L, L)] = a * w
                    rowf[pl.ds(kk * 2 * L + L, L)] = b * w
                return 0
            lax.fori_loop(0, B, scale, 0)
            pltpu.sync_copy(rowsf, slab.at[wll], add=True)

        resentinel()
        myrow = sid * SH

        def chunk_body(ci, _):
            chunk = NC * ci + cid
            lo = chunk * R

            zero_rowsf()
            pltpu.sync_copy(rowsf, slab.at[pl.ds(myrow, B)])
            pltpu.sync_copy(rowsf.at[pl.ds(0, SH - B)],
                            slab.at[pl.ds(myrow + B, SH - B)])

            def zden(i, _):
                den0[pl.ds(i * L, L)] = zf
                den1[pl.ds(i * L, L)] = zf
                return 0
            lax.fori_loop(0, R // L, zden, 0)
            plsc.subcore_barrier()

            def block(bi, carry):
                eoff = base + bi * EBLK
                pltpu.sync_copy(src_hbm.at[pl.ds(eoff, EBLK)], eb_src)
                pltpu.sync_copy(dst_hbm.at[pl.ds(eoff, EBLK)], eb_dst)
                pltpu.sync_copy(w0_hbm.at[pl.ds(eoff, EBLK)], eb_w0)
                pltpu.sync_copy(w1_hbm.at[pl.ds(eoff, EBLK)], eb_w1)

                def group(g, carry):
                    cnt, fb, warm = carry
                    s16 = eb_src[pl.ds(g * L, L)]
                    d16 = eb_dst[pl.ds(g * L, L)]
                    w016 = eb_w0[pl.ds(g * L, L)]
                    w116 = eb_w1[pl.ds(g * L, L)]
                    m = (d16 >= lo) & (d16 < lo + R)
                    loc16 = d16 - lo
                    plsc.addupdate_scatter(den0, [loc16], w016, mask=m)
                    plsc.addupdate_scatter(den1, [loc16], w116, mask=m)
                    plsc.store_compressed(st_src.at[pl.ds(cnt, L)], s16, mask=m)
                    plsc.store_compressed(st_loc.at[pl.ds(cnt, L)], loc16, mask=m)
                    plsc.store_compressed(st_w0.at[pl.ds(cnt, L)], w016, mask=m)
                    plsc.store_compressed(st_w1.at[pl.ds(cnt, L)], w116, mask=m)
                    cnt = cnt + plsc.all_reduce_population_count(m)[0]
                    full = cnt > B - L
                    for p in range(2):
                        @pl.when(full & (fb == p))
                        def _():
                            snapshot(p)
                            start_gather(p)

                            @pl.when(warm == 1)
                            def _():
                                drain(1 - p)
                    cnt = jnp.where(full, jnp.int32(0), cnt)
                    warm = jnp.where(full, jnp.int32(1), warm)
                    fb = jnp.where(full, 1 - fb, fb)
                    return (cnt, fb, warm)

                return lax.fori_loop(0, EBLK_G, group, carry)

            cnt, fb, warm = lax.fori_loop(
                0, NBLK, block,
                (jnp.int32(0), jnp.int32(0), jnp.int32(0)))

            for p in range(2):
                @pl.when((warm == 1) & (fb == p))
                def _():
                    drain(1 - p)
            for p in range(2):
                @pl.when(fb == p)
                def _():
                    snapshot(p)
                    start_gather(p)
                    drain(p)
            plsc.subcore_barrier()

            pltpu.sync_copy(den0, dall0.at[sid])
            pltpu.sync_copy(den1, dall1.at[sid])
            plsc.subcore_barrier()

            def dfetch(t, _):
                pltpu.sync_copy(dall0.at[t, pl.ds(myrow, SH)], dc0.at[t])
                pltpu.sync_copy(dall1.at[t, pl.ds(myrow, SH)], dc1.at[t])
                return 0
            lax.fori_loop(0, NS, dfetch, 0)

            def dsum(t, _):
                for q in range(SH // L):
                    dc0[0, pl.ds(q * L, L)] = (dc0[0, pl.ds(q * L, L)]
                                               + dc0[t, pl.ds(q * L, L)])
                    dc1[0, pl.ds(q * L, L)] = (dc1[0, pl.ds(q * L, L)]
                                               + dc1[t, pl.ds(q * L, L)])
                return 0
            lax.fori_loop(1, NS, dsum, 0)

            def wblk(blk, _):
                pltpu.sync_copy(slab.at[pl.ds(myrow + blk * L, L)],
                                rowsf.at[pl.ds(0, L)])
                inv0_v[...] = 1.0 / jnp.maximum(dc0[0, pl.ds(blk * L, L)], 1e-16)
                inv1_v[...] = 1.0 / jnp.maximum(dc1[0, pl.ds(blk * L, L)], 1e-16)

                def wrow(e, _):
                    ev = jnp.full((L,), e, jnp.int32)
                    i0 = plsc.load_gather(inv0_v, [ev])
                    i1 = plsc.load_gather(inv1_v, [ev])
                    row = rowsf.at[e]
                    for kk in range(KV // 2):
                        row[pl.ds(kk * L, L)] = (row[pl.ds(kk * L, L)] * i0
                                                 + bias_v[pl.ds(kk * L, L)])
                    for kk in range(KV // 2, KV):
                        row[pl.ds(kk * L, L)] = (row[pl.ds(kk * L, L)] * i1
                                                 + bias_v[pl.ds(kk * L, L)])
                    return 0
                lax.fori_loop(0, L, wrow, 0)
                pltpu.sync_copy(rowsf.at[pl.ds(0, L)],
                                out_hbm.at[pl.ds(lo + myrow + blk * L, L)])
                return 0
            lax.fori_loop(0, SH // L, wblk, 0)
            return 0

        lax.fori_loop(0, NCHUNK // NC, chunk_body, 0)

    return k(xp16, src, dst, w0, w1, bias)


def kernel(x, undirected_edges_small_middle, W, att_src, att_dst, bias):
    N = x.shape[0]
    H, C = att_src.shape
    HC = H * C

    A = jnp.zeros((HC, 128), jnp.float32)
    A = A.at[0:C, 0].set(att_src[0]).at[C:HC, 1].set(att_src[1])
    A = A.at[0:C, 2].set(att_dst[0]).at[C:HC, 3].set(att_dst[1])

    perm = np.empty((HC,), np.int32)
    for kk in range(HC // 32):
        for j in range(16):
            perm[32 * kk + 2 * j] = 32 * kk + j
            perm[32 * kk + 2 * j + 1] = 32 * kk + 16 + j
    Wp = W[:, perm]

    xp, a_all, xp16 = _tc_project(x, W, Wp, A)
    del xp
    a4 = a_all[:, :4].reshape(-1)

    ei = undirected_edges_small_middle.astype(jnp.int32)
    E = ei.shape[1]
    loops = jnp.arange(N, dtype=jnp.int32)
    e_tot = E + N
    e_pad = -(-e_tot // (NC * NS * L)) * (NC * NS * L)
    npad = e_pad - e_tot
    pad_src = (jnp.arange(npad, dtype=jnp.int32) * 613) % N
    pad_dst = jnp.full((npad,), jnp.int32(1 << 30))
    src = jnp.concatenate([ei[0], loops, pad_src])
    dst = jnp.concatenate([ei[1], loops, pad_dst])

    w0, w1 = _sc_weights(src, dst, a4, N)
    out_pad = _sc_scatter(xp16, src, dst, w0, w1, bias)
    return out_pad[:N]


# speedup vs baseline: 1.6728x; 1.6728x over previous
import functools

import jax
import jax.numpy as jnp
from jax import lax
from jax.experimental import pallas as pl
from jax.experimental.pallas import tpu as pltpu
from jax.experimental.pallas import tpu_sc as plsc

NC = 2
NS = 16
L = 16

R = 1280
NCHUNK = 8
B = 64


def _tc_project(x, W, A):
    N, IN = x.shape
    HC = W.shape[1]
    BLK = 2000 if N % 2000 == 0 else N

    def body(x_ref, w_ref, a_ref, xp_ref, aall_ref):
        xp = jnp.dot(x_ref[...], w_ref[...], preferred_element_type=jnp.float32)
        xp_ref[...] = xp
        aall_ref[...] = jnp.dot(xp, a_ref[...], preferred_element_type=jnp.float32)

    return pl.pallas_call(
        body,
        grid=(N // BLK,),
        in_specs=[
            pl.BlockSpec((BLK, IN), lambda i: (i, 0)),
            pl.BlockSpec((IN, HC), lambda i: (0, 0)),
            pl.BlockSpec((HC, 128), lambda i: (0, 0)),
        ],
        out_specs=[
            pl.BlockSpec((BLK, HC), lambda i: (i, 0)),
            pl.BlockSpec((BLK, 128), lambda i: (i, 0)),
        ],
        out_shape=[
            jax.ShapeDtypeStruct((N, HC), jnp.float32),
            jax.ShapeDtypeStruct((N, 128), jnp.float32),
        ],
    )(x, W, A)


def _sc_weights(src, dst, a4, n_nodes):
    E_pad = src.shape[0]
    SPAN = E_pad // (NC * NS)
    N = n_nodes

    mesh = plsc.VectorSubcoreMesh(
        core_axis_name="c", subcore_axis_name="s", num_cores=NC, num_subcores=NS)

    @functools.partial(
        pl.kernel,
        mesh=mesh,
        compiler_params=pltpu.CompilerParams(
            use_tc_tiling_on_sc=False, needs_layout_passes=False),
        out_type=[
            jax.ShapeDtypeStruct((E_pad,), jnp.float32),
            jax.ShapeDtypeStruct((E_pad,), jnp.float32),
        ],
        scratch_types=[
            pltpu.VMEM((N * 4,), jnp.float32),
            pltpu.VMEM((SPAN,), jnp.int32),
            pltpu.VMEM((SPAN,), jnp.int32),
            pltpu.VMEM((SPAN,), jnp.float32),
            pltpu.VMEM((SPAN,), jnp.float32),
        ],
    )
    def k(src_hbm, dst_hbm, a4_hbm, w0_hbm, w1_hbm,
          a4_v, src_v, dst_v, w0_v, w1_v):
        cid = lax.axis_index("c")
        sid = lax.axis_index("s")
        base = (cid * NS + sid) * SPAN
        pltpu.sync_copy(a4_hbm, a4_v)
        pltpu.sync_copy(src_hbm.at[pl.ds(base, SPAN)], src_v)
        pltpu.sync_copy(dst_hbm.at[pl.ds(base, SPAN)], dst_v)

        def grp(g, _):
            s16 = src_v[pl.ds(g * L, L)]
            d16 = dst_v[pl.ds(g * L, L)]
            valid = d16 < jnp.int32(N)
            s4 = s16 * 4
            d4 = d16 * 4
            as0 = plsc.load_gather(a4_v, [s4], mask=valid)
            as1 = plsc.load_gather(a4_v, [s4 + 1], mask=valid)
            ad0 = plsc.load_gather(a4_v, [d4 + 2], mask=valid)
            ad1 = plsc.load_gather(a4_v, [d4 + 3], mask=valid)
            al0 = as0 + ad0
            al1 = as1 + ad1
            al0 = jnp.where(al0 > 0, al0, 0.2 * al0)
            al1 = jnp.where(al1 > 0, al1, 0.2 * al1)
            w0_v[pl.ds(g * L, L)] = jnp.where(valid, jnp.exp(al0), 0.0)
            w1_v[pl.ds(g * L, L)] = jnp.where(valid, jnp.exp(al1), 0.0)
            return 0

        lax.fori_loop(0, SPAN // L, grp, 0)
        pltpu.sync_copy(w0_v, w0_hbm.at[pl.ds(base, SPAN)])
        pltpu.sync_copy(w1_v, w1_hbm.at[pl.ds(base, SPAN)])

    return k(src, dst, a4)


def _sc_scatter(xp, src, dst, w0, w1, bias):
    N, HC = xp.shape
    KV = HC // L
    E_pad = src.shape[0]
    SPAN = E_pad // NS
    GROUPS = SPAN // L
    EBLK_G = 74
    EBLK = EBLK_G * L
    NBLK = GROUPS // EBLK_G
    NPAD = R * NCHUNK
    SH = R // NS

    assert GROUPS % EBLK_G == 0

    mesh = plsc.VectorSubcoreMesh(
        core_axis_name="c", subcore_axis_name="s", num_cores=NC, num_subcores=NS)

    buf_types = []
    for _ in range(2):
        buf_types += [
            pltpu.VMEM((B, HC), jnp.float32),
            pltpu.VMEM((B,), jnp.int32),
            pltpu.VMEM((B,), jnp.int32),
            pltpu.VMEM((B,), jnp.float32),
            pltpu.VMEM((B,), jnp.float32),
            pltpu.SemaphoreType.DMA,
        ]

    @functools.partial(
        pl.kernel,
        mesh=mesh,
        compiler_params=pltpu.CompilerParams(
            use_tc_tiling_on_sc=False, needs_layout_passes=False),
        out_type=jax.ShapeDtypeStruct((NPAD, HC), jnp.float32),
        scratch_types=[
            pltpu.VMEM_SHARED((R, HC), jnp.float32),
            pltpu.VMEM_SHARED((NS, R), jnp.float32),
            pltpu.VMEM_SHARED((NS, R), jnp.float32),
            pltpu.VMEM((EBLK,), jnp.int32),
            pltpu.VMEM((EBLK,), jnp.int32),
            pltpu.VMEM((EBLK,), jnp.float32),
            pltpu.VMEM((EBLK,), jnp.float32),
            pltpu.VMEM((R,), jnp.float32),
            pltpu.VMEM((R,), jnp.float32),
            pltpu.VMEM((HC,), jnp.float32),
            pltpu.VMEM((NS, SH), jnp.float32),
            pltpu.VMEM((NS, SH), jnp.float32),
            pltpu.VMEM((L,), jnp.float32),
            pltpu.VMEM((L,), jnp.float32),
            pltpu.VMEM((B,), jnp.int32),
            pltpu.VMEM((B,), jnp.int32),
            pltpu.VMEM((B,), jnp.float32),
            pltpu.VMEM((B,), jnp.float32),
        ] + buf_types,
    )
    def k(xp_hbm, src_hbm, dst_hbm, w0_hbm, w1_hbm, bias_hbm, out_hbm,
          slab, dall0, dall1, eb_src, eb_dst, eb_w0, eb_w1,
          den0, den1, bias_v, dc0, dc1, inv0_v, inv1_v,
          st_src, st_loc, st_w0, st_w1,
          rows_a, wls_a, wll_a, ww0_a, ww1_a, gsem_a,
          rows_b, wls_b, wll_b, ww0_b, ww1_b, gsem_b):
        cid = lax.axis_index("c")
        sid = lax.axis_index("s")
        base = sid * SPAN
        pltpu.sync_copy(bias_hbm, bias_v)

        bufs = ((rows_a, wls_a, wll_a, ww0_a, ww1_a, gsem_a),
                (rows_b, wls_b, wll_b, ww0_b, ww1_b, gsem_b))

        zf = jnp.zeros((L,), jnp.float32)
        iot = lax.iota(jnp.int32, L)

        def resentinel():
            for b in range(B // L):
                st_src[pl.ds(b * L, L)] = (iot * 601 + b * 977) % N
                st_loc[pl.ds(b * L, L)] = (iot * 37 + b * 53) % R
                st_w0[pl.ds(b * L, L)] = zf
                st_w1[pl.ds(b * L, L)] = zf

        def snapshot(p):
            _, wls, wll, ww0, ww1, _ = bufs[p]
            for b in range(B // L):
                sl = pl.ds(b * L, L)
                wls[sl] = st_src[sl]
                wll[sl] = st_loc[sl]
                ww0[sl] = st_w0[sl]
                ww1[sl] = st_w1[sl]
            resentinel()

        def zero_rowsf():
            rowz = bufs[0][0]

            def zrow(e, _):
                row = rowz.at[e]
                for kk in range(KV):
                    row[pl.ds(kk * L, L)] = zf
                return 0
            lax.fori_loop(0, B, zrow, 0)

        def start_gather(p):
            rows, wls = bufs[p][0], bufs[p][1]
            pltpu.make_async_copy(xp_hbm.at[wls], rows, bufs[p][5]).start()

        def drain(p):
            rows, wls, wll, ww0, ww1, gsem = bufs[p]
            pltpu.make_async_copy(xp_hbm.at[wls], rows, gsem).wait()

            def scale(e, _):
                ev = jnp.full((L,), e, jnp.int32)
                wv0 = plsc.load_gather(ww0, [ev])
                wv1 = plsc.load_gather(ww1, [ev])
                row = rows.at[e]
                for kk in range(KV // 2):
                    row[pl.ds(kk * L, L)] = row[pl.ds(kk * L, L)] * wv0
                for kk in range(KV // 2, KV):
                    row[pl.ds(kk * L, L)] = row[pl.ds(kk * L, L)] * wv1
                return 0
            lax.fori_loop(0, B, scale, 0)
            pltpu.sync_copy(rows, slab.at[wll], add=True)

        resentinel()
        myrow = sid * SH

        def chunk_body(ci, _):
            chunk = NC * ci + cid
            lo = chunk * R

            zero_rowsf()
            rowz = bufs[0][0]
            pltpu.sync_copy(rowz, slab.at[pl.ds(myrow, B)])
            pltpu.sync_copy(rowz.at[pl.ds(0, SH - B)],
                            slab.at[pl.ds(myrow + B, SH - B)])

            def zden(i, _):
                den0[pl.ds(i * L, L)] = zf
                den1[pl.ds(i * L, L)] = zf
                return 0
            lax.fori_loop(0, R // L, zden, 0)
            plsc.subcore_barrier()

            def block(bi, carry):
                eoff = base + bi * EBLK
                pltpu.sync_copy(src_hbm.at[pl.ds(eoff, EBLK)], eb_src)
                pltpu.sync_copy(dst_hbm.at[pl.ds(eoff, EBLK)], eb_dst)
                pltpu.sync_copy(w0_hbm.at[pl.ds(eoff, EBLK)], eb_w0)
                pltpu.sync_copy(w1_hbm.at[pl.ds(eoff, EBLK)], eb_w1)

                def group(g, carry):
                    cnt, fb, warm = carry
                    s16 = eb_src[pl.ds(g * L, L)]
                    d16 = eb_dst[pl.ds(g * L, L)]
                    w016 = eb_w0[pl.ds(g * L, L)]
                    w116 = eb_w1[pl.ds(g * L, L)]
                    m = (d16 >= lo) & (d16 < lo + R)
                    loc16 = d16 - lo
                    plsc.addupdate_scatter(den0, [loc16], w016, mask=m)
                    plsc.addupdate_scatter(den1, [loc16], w116, mask=m)
                    plsc.store_compressed(st_src.at[pl.ds(cnt, L)], s16, mask=m)
                    plsc.store_compressed(st_loc.at[pl.ds(cnt, L)], loc16, mask=m)
                    plsc.store_compressed(st_w0.at[pl.ds(cnt, L)], w016, mask=m)
                    plsc.store_compressed(st_w1.at[pl.ds(cnt, L)], w116, mask=m)
                    cnt = cnt + plsc.all_reduce_population_count(m)[0]
                    full = cnt > B - L
                    for p in range(2):
                        @pl.when(full & (fb == p))
                        def _():
                            snapshot(p)
                            start_gather(p)

                            @pl.when(warm == 1)
                            def _():
                                drain(1 - p)
                    cnt = jnp.where(full, jnp.int32(0), cnt)
                    warm = jnp.where(full, jnp.int32(1), warm)
                    fb = jnp.where(full, 1 - fb, fb)
                    return (cnt, fb, warm)

                return lax.fori_loop(0, EBLK_G, group, carry)

            cnt, fb, warm = lax.fori_loop(
                0, NBLK, block,
                (jnp.int32(0), jnp.int32(0), jnp.int32(0)))

            for p in range(2):
                @pl.when((warm == 1) & (fb == p))
                def _():
                    drain(1 - p)
            for p in range(2):
                @pl.when(fb == p)
                def _():
                    snapshot(p)
                    start_gather(p)
                    drain(p)
            plsc.subcore_barrier()

            pltpu.sync_copy(den0, dall0.at[sid])
            pltpu.sync_copy(den1, dall1.at[sid])
            plsc.subcore_barrier()

            def dfetch(t, _):
                pltpu.sync_copy(dall0.at[t, pl.ds(myrow, SH)], dc0.at[t])
                pltpu.sync_copy(dall1.at[t, pl.ds(myrow, SH)], dc1.at[t])
                return 0
            lax.fori_loop(0, NS, dfetch, 0)

            def dsum(t, _):
                for q in range(SH // L):
                    dc0[0, pl.ds(q * L, L)] = (dc0[0, pl.ds(q * L, L)]
                                               + dc0[t, pl.ds(q * L, L)])
                    dc1[0, pl.ds(q * L, L)] = (dc1[0, pl.ds(q * L, L)]
                                               + dc1[t, pl.ds(q * L, L)])
                return 0
            lax.fori_loop(1, NS, dsum, 0)

            def wblk(blk, _):
                wrows = bufs[0][0]
                pltpu.sync_copy(slab.at[pl.ds(myrow + blk * L, L)],
                                wrows.at[pl.ds(0, L)])
                inv0_v[...] = 1.0 / jnp.maximum(dc0[0, pl.ds(blk * L, L)], 1e-16)
                inv1_v[...] = 1.0 / jnp.maximum(dc1[0, pl.ds(blk * L, L)], 1e-16)

                def wrow(e, _):
                    ev = jnp.full((L,), e, jnp.int32)
                    i0 = plsc.load_gather(inv0_v, [ev])
                    i1 = plsc.load_gather(inv1_v, [ev])
                    row = wrows.at[e]
                    for kk in range(KV // 2):
                        row[pl.ds(kk * L, L)] = (row[pl.ds(kk * L, L)] * i0
                                                 + bias_v[pl.ds(kk * L, L)])
                    for kk in range(KV // 2, KV):
                        row[pl.ds(kk * L, L)] = (row[pl.ds(kk * L, L)] * i1
                                                 + bias_v[pl.ds(kk * L, L)])
                    return 0
                lax.fori_loop(0, L, wrow, 0)
                pltpu.sync_copy(wrows.at[pl.ds(0, L)],
                                out_hbm.at[pl.ds(lo + myrow + blk * L, L)])
                return 0
            lax.fori_loop(0, SH // L, wblk, 0)
            return 0

        lax.fori_loop(0, NCHUNK // NC, chunk_body, 0)

    return k(xp, src, dst, w0, w1, bias)


def kernel(x, undirected_edges_small_middle, W, att_src, att_dst, bias):
    N = x.shape[0]
    H, C = att_src.shape
    HC = H * C

    A = jnp.zeros((HC, 128), jnp.float32)
    A = A.at[0:C, 0].set(att_src[0]).at[C:HC, 1].set(att_src[1])
    A = A.at[0:C, 2].set(att_dst[0]).at[C:HC, 3].set(att_dst[1])

    xp, a_all = _tc_project(x, W, A)
    a4 = a_all[:, :4].reshape(-1)

    ei = undirected_edges_small_middle.astype(jnp.int32)
    E = ei.shape[1]
    loops = jnp.arange(N, dtype=jnp.int32)
    e_tot = E + N
    e_pad = -(-e_tot // (NC * NS * L)) * (NC * NS * L)
    npad = e_pad - e_tot
    pad_src = (jnp.arange(npad, dtype=jnp.int32) * 613) % N
    pad_dst = jnp.full((npad,), jnp.int32(1 << 30))
    src = jnp.concatenate([ei[0], loops, pad_src])
    dst = jnp.concatenate([ei[1], loops, pad_dst])

    w0, w1 = _sc_weights(src, dst, a4, N)
    out_pad = _sc_scatter(xp, src, dst, w0, w1, bias)
    return out_pad[:N]


# speedup vs baseline: 1.7495x; 1.0458x over previous
import functools

import jax
import jax.numpy as jnp
from jax import lax
from jax.experimental import pallas as pl
from jax.experimental.pallas import tpu as pltpu
from jax.experimental.pallas import tpu_sc as plsc

NC = 2
NS = 16
L = 16

R = 1280
NCHUNK = 8
B = 64


def _tc_project(x, W, A):
    N, IN = x.shape
    HC = W.shape[1]
    BLK = 2000 if N % 2000 == 0 else N

    def body(x_ref, w_ref, a_ref, xp_ref, aall_ref):
        xp = jnp.dot(x_ref[...], w_ref[...], preferred_element_type=jnp.float32)
        xp_ref[...] = xp
        aall_ref[...] = jnp.dot(xp, a_ref[...], preferred_element_type=jnp.float32)

    return pl.pallas_call(
        body,
        grid=(N // BLK,),
        in_specs=[
            pl.BlockSpec((BLK, IN), lambda i: (i, 0)),
            pl.BlockSpec((IN, HC), lambda i: (0, 0)),
            pl.BlockSpec((HC, 128), lambda i: (0, 0)),
        ],
        out_specs=[
            pl.BlockSpec((BLK, HC), lambda i: (i, 0)),
            pl.BlockSpec((BLK, 128), lambda i: (i, 0)),
        ],
        out_shape=[
            jax.ShapeDtypeStruct((N, HC), jnp.float32),
            jax.ShapeDtypeStruct((N, 128), jnp.float32),
        ],
    )(x, W, A)


def _sc_weights(src, dst, a4, n_nodes):
    E_pad = src.shape[0]
    SPAN = E_pad // (NC * NS)
    N = n_nodes

    mesh = plsc.VectorSubcoreMesh(
        core_axis_name="c", subcore_axis_name="s", num_cores=NC, num_subcores=NS)

    @functools.partial(
        pl.kernel,
        mesh=mesh,
        compiler_params=pltpu.CompilerParams(
            use_tc_tiling_on_sc=False, needs_layout_passes=False),
        out_type=[
            jax.ShapeDtypeStruct((E_pad,), jnp.float32),
            jax.ShapeDtypeStruct((E_pad,), jnp.float32),
        ],
        scratch_types=[
            pltpu.VMEM((N * 4,), jnp.float32),
            pltpu.VMEM((SPAN,), jnp.int32),
            pltpu.VMEM((SPAN,), jnp.int32),
            pltpu.VMEM((SPAN,), jnp.float32),
            pltpu.VMEM((SPAN,), jnp.float32),
        ],
    )
    def k(src_hbm, dst_hbm, a4_hbm, w0_hbm, w1_hbm,
          a4_v, src_v, dst_v, w0_v, w1_v):
        cid = lax.axis_index("c")
        sid = lax.axis_index("s")
        base = (cid * NS + sid) * SPAN
        pltpu.sync_copy(a4_hbm, a4_v)
        pltpu.sync_copy(src_hbm.at[pl.ds(base, SPAN)], src_v)
        pltpu.sync_copy(dst_hbm.at[pl.ds(base, SPAN)], dst_v)

        def grp(g, _):
            s16 = src_v[pl.ds(g * L, L)]
            d16 = dst_v[pl.ds(g * L, L)]
            valid = d16 < jnp.int32(N)
            s4 = s16 * 4
            d4 = d16 * 4
            as0 = plsc.load_gather(a4_v, [s4], mask=valid)
            as1 = plsc.load_gather(a4_v, [s4 + 1], mask=valid)
            ad0 = plsc.load_gather(a4_v, [d4 + 2], mask=valid)
            ad1 = plsc.load_gather(a4_v, [d4 + 3], mask=valid)
            al0 = as0 + ad0
            al1 = as1 + ad1
            al0 = jnp.where(al0 > 0, al0, 0.2 * al0)
            al1 = jnp.where(al1 > 0, al1, 0.2 * al1)
            w0_v[pl.ds(g * L, L)] = jnp.where(valid, jnp.exp(al0), 0.0)
            w1_v[pl.ds(g * L, L)] = jnp.where(valid, jnp.exp(al1), 0.0)
            return 0

        lax.fori_loop(0, SPAN // L, grp, 0)
        pltpu.sync_copy(w0_v, w0_hbm.at[pl.ds(base, SPAN)])
        pltpu.sync_copy(w1_v, w1_hbm.at[pl.ds(base, SPAN)])

    return k(src, dst, a4)


def _sc_scatter(xp, src, dst, w0, w1, bias):
    N, HC = xp.shape
    KV = HC // L
    E_pad = src.shape[0]
    SPAN = E_pad // NS
    GROUPS = SPAN // L
    EBLK_G = 74
    EBLK = EBLK_G * L
    NBLK = GROUPS // EBLK_G
    NPAD = R * NCHUNK
    SH = R // NS

    assert GROUPS % EBLK_G == 0

    mesh = plsc.VectorSubcoreMesh(
        core_axis_name="c", subcore_axis_name="s", num_cores=NC, num_subcores=NS)

    buf_types = []
    for _ in range(2):
        buf_types += [
            pltpu.VMEM((B, HC), jnp.float32),
            pltpu.VMEM((B,), jnp.int32),
            pltpu.VMEM((B,), jnp.int32),
            pltpu.VMEM((B,), jnp.float32),
            pltpu.VMEM((B,), jnp.float32),
            pltpu.VMEM((B,), jnp.int32),
            pltpu.SemaphoreType.DMA,
            pltpu.SemaphoreType.DMA,
        ]

    @functools.partial(
        pl.kernel,
        mesh=mesh,
        compiler_params=pltpu.CompilerParams(
            use_tc_tiling_on_sc=False, needs_layout_passes=False),
        out_type=jax.ShapeDtypeStruct((NPAD, HC), jnp.float32),
        scratch_types=[
            pltpu.VMEM_SHARED((R, HC), jnp.float32),
            pltpu.VMEM_SHARED((NS, R), jnp.float32),
            pltpu.VMEM_SHARED((NS, R), jnp.float32),
            pltpu.VMEM((EBLK,), jnp.int32),
            pltpu.VMEM((EBLK,), jnp.int32),
            pltpu.VMEM((EBLK,), jnp.float32),
            pltpu.VMEM((EBLK,), jnp.float32),
            pltpu.VMEM((R,), jnp.float32),
            pltpu.VMEM((R,), jnp.float32),
            pltpu.VMEM((HC,), jnp.float32),
            pltpu.VMEM((NS, SH), jnp.float32),
            pltpu.VMEM((NS, SH), jnp.float32),
            pltpu.VMEM((L,), jnp.float32),
            pltpu.VMEM((L,), jnp.float32),
            pltpu.VMEM((B,), jnp.int32),
            pltpu.VMEM((B,), jnp.int32),
            pltpu.VMEM((B,), jnp.float32),
            pltpu.VMEM((B,), jnp.float32),
        ] + buf_types,
    )
    def k(xp_hbm, src_hbm, dst_hbm, w0_hbm, w1_hbm, bias_hbm, out_hbm,
          slab, dall0, dall1, eb_src, eb_dst, eb_w0, eb_w1,
          den0, den1, bias_v, dc0, dc1, inv0_v, inv1_v,
          st_src, st_loc, st_w0, st_w1,
          rows_a, wls_a, wll_a, ww0_a, ww1_a, sloc_a, gsem_a, ssem_a,
          rows_b, wls_b, wll_b, ww0_b, ww1_b, sloc_b, gsem_b, ssem_b):
        cid = lax.axis_index("c")
        sid = lax.axis_index("s")
        base = sid * SPAN
        pltpu.sync_copy(bias_hbm, bias_v)

        bufs = ((rows_a, wls_a, wll_a, ww0_a, ww1_a, sloc_a, gsem_a, ssem_a),
                (rows_b, wls_b, wll_b, ww0_b, ww1_b, sloc_b, gsem_b, ssem_b))

        zf = jnp.zeros((L,), jnp.float32)
        iot = lax.iota(jnp.int32, L)

        def resentinel():
            for b in range(B // L):
                st_src[pl.ds(b * L, L)] = (iot * 601 + b * 977) % N
                st_loc[pl.ds(b * L, L)] = (iot * 37 + b * 53) % R
                st_w0[pl.ds(b * L, L)] = zf
                st_w1[pl.ds(b * L, L)] = zf

        def snapshot(p):
            _, wls, wll, ww0, ww1, _, _, _ = bufs[p]
            for b in range(B // L):
                sl = pl.ds(b * L, L)
                wls[sl] = st_src[sl]
                wll[sl] = st_loc[sl]
                ww0[sl] = st_w0[sl]
                ww1[sl] = st_w1[sl]
            resentinel()

        def zero_rowsf():
            rowz = bufs[0][0]

            def zrow(e, _):
                row = rowz.at[e]
                for kk in range(KV):
                    row[pl.ds(kk * L, L)] = zf
                return 0
            lax.fori_loop(0, B, zrow, 0)

        def start_gather(p):
            rows, wls = bufs[p][0], bufs[p][1]
            pltpu.make_async_copy(xp_hbm.at[wls], rows, bufs[p][6]).start()

        def wait_scatter(p):
            rows, sloc, ssem = bufs[p][0], bufs[p][5], bufs[p][7]
            pltpu.make_async_copy(rows, slab.at[sloc], ssem).wait()

        def drain(p):
            rows, wls, wll, ww0, ww1, sloc, gsem, ssem = bufs[p]
            pltpu.make_async_copy(xp_hbm.at[wls], rows, gsem).wait()
            for b in range(B // L):
                sloc[pl.ds(b * L, L)] = wll[pl.ds(b * L, L)]

            def scale(e, _):
                ev = jnp.full((L,), e, jnp.int32)
                wv0 = plsc.load_gather(ww0, [ev])
                wv1 = plsc.load_gather(ww1, [ev])
                row = rows.at[e]
                for kk in range(KV // 2):
                    row[pl.ds(kk * L, L)] = row[pl.ds(kk * L, L)] * wv0
                for kk in range(KV // 2, KV):
                    row[pl.ds(kk * L, L)] = row[pl.ds(kk * L, L)] * wv1
                return 0
            lax.fori_loop(0, B, scale, 0)
            pltpu.make_async_copy(rows, slab.at[sloc], ssem).start(add=True)

        resentinel()
        myrow = sid * SH

        def chunk_body(ci, _):
            chunk = NC * ci + cid
            lo = chunk * R

            zero_rowsf()
            rowz = bufs[0][0]
            pltpu.sync_copy(rowz, slab.at[pl.ds(myrow, B)])
            pltpu.sync_copy(rowz.at[pl.ds(0, SH - B)],
                            slab.at[pl.ds(myrow + B, SH - B)])

            def zden(i, _):
                den0[pl.ds(i * L, L)] = zf
                den1[pl.ds(i * L, L)] = zf
                return 0
            lax.fori_loop(0, R // L, zden, 0)
            plsc.subcore_barrier()

            def block(bi, carry):
                eoff = base + bi * EBLK
                pltpu.sync_copy(src_hbm.at[pl.ds(eoff, EBLK)], eb_src)
                pltpu.sync_copy(dst_hbm.at[pl.ds(eoff, EBLK)], eb_dst)
                pltpu.sync_copy(w0_hbm.at[pl.ds(eoff, EBLK)], eb_w0)
                pltpu.sync_copy(w1_hbm.at[pl.ds(eoff, EBLK)], eb_w1)

                def group(g, carry):
                    cnt, fb, warm, sp0, sp1 = carry
                    sp = (sp0, sp1)
                    s16 = eb_src[pl.ds(g * L, L)]
                    d16 = eb_dst[pl.ds(g * L, L)]
                    w016 = eb_w0[pl.ds(g * L, L)]
                    w116 = eb_w1[pl.ds(g * L, L)]
                    m = (d16 >= lo) & (d16 < lo + R)
                    loc16 = d16 - lo
                    plsc.addupdate_scatter(den0, [loc16], w016, mask=m)
                    plsc.addupdate_scatter(den1, [loc16], w116, mask=m)
                    plsc.store_compressed(st_src.at[pl.ds(cnt, L)], s16, mask=m)
                    plsc.store_compressed(st_loc.at[pl.ds(cnt, L)], loc16, mask=m)
                    plsc.store_compressed(st_w0.at[pl.ds(cnt, L)], w016, mask=m)
                    plsc.store_compressed(st_w1.at[pl.ds(cnt, L)], w116, mask=m)
                    cnt = cnt + plsc.all_reduce_population_count(m)[0]
                    full = cnt > B - L
                    for p in range(2):
                        @pl.when(full & (fb == p))
                        def _():
                            @pl.when(sp[p] == 1)
                            def _():
                                wait_scatter(p)
                            snapshot(p)
                            start_gather(p)

                            @pl.when(warm == 1)
                            def _():
                                drain(1 - p)
                    sp0 = jnp.where(full, jnp.where(fb == 0, 0, warm), sp0)
                    sp1 = jnp.where(full, jnp.where(fb == 1, 0, warm), sp1)
                    cnt = jnp.where(full, jnp.int32(0), cnt)
                    warm = jnp.where(full, jnp.int32(1), warm)
                    fb = jnp.where(full, 1 - fb, fb)
                    return (cnt, fb, warm, sp0, sp1)

                return lax.fori_loop(0, EBLK_G, group, carry)

            cnt, fb, warm, sp0, sp1 = lax.fori_loop(
                0, NBLK, block,
                (jnp.int32(0), jnp.int32(0), jnp.int32(0),
                 jnp.int32(0), jnp.int32(0)))
            sp = (sp0, sp1)

            for p in range(2):
                @pl.when((warm == 1) & (fb == p))
                def _():
                    drain(1 - p)
            for p in range(2):
                @pl.when(fb == p)
                def _():
                    @pl.when(sp[p] == 1)
                    def _():
                        wait_scatter(p)
                    snapshot(p)
                    start_gather(p)
                    drain(p)
                    wait_scatter(p)
            for p in range(2):
                @pl.when((warm == 1) & (fb == p))
                def _():
                    wait_scatter(1 - p)
            plsc.subcore_barrier()

            pltpu.sync_copy(den0, dall0.at[sid])
            pltpu.sync_copy(den1, dall1.at[sid])
            plsc.subcore_barrier()

            def dfetch(t, _):
                pltpu.sync_copy(dall0.at[t, pl.ds(myrow, SH)], dc0.at[t])
                pltpu.sync_copy(dall1.at[t, pl.ds(myrow, SH)], dc1.at[t])
                return 0
            lax.fori_loop(0, NS, dfetch, 0)

            def dsum(t, _):
                for q in range(SH // L):
                    dc0[0, pl.ds(q * L, L)] = (dc0[0, pl.ds(q * L, L)]
                                               + dc0[t, pl.ds(q * L, L)])
                    dc1[0, pl.ds(q * L, L)] = (dc1[0, pl.ds(q * L, L)]
                                               + dc1[t, pl.ds(q * L, L)])
                return 0
            lax.fori_loop(1, NS, dsum, 0)

            def wblk(blk, _):
                wrows = bufs[0][0]
                pltpu.sync_copy(slab.at[pl.ds(myrow + blk * L, L)],
                                wrows.at[pl.ds(0, L)])
                inv0_v[...] = 1.0 / jnp.maximum(dc0[0, pl.ds(blk * L, L)], 1e-16)
                inv1_v[...] = 1.0 / jnp.maximum(dc1[0, pl.ds(blk * L, L)], 1e-16)

                def wrow(e, _):
                    ev = jnp.full((L,), e, jnp.int32)
                    i0 = plsc.load_gather(inv0_v, [ev])
                    i1 = plsc.load_gather(inv1_v, [ev])
                    row = wrows.at[e]
                    for kk in range(KV // 2):
                        row[pl.ds(kk * L, L)] = (row[pl.ds(kk * L, L)] * i0
                                                 + bias_v[pl.ds(kk * L, L)])
                    for kk in range(KV // 2, KV):
                        row[pl.ds(kk * L, L)] = (row[pl.ds(kk * L, L)] * i1
                                                 + bias_v[pl.ds(kk * L, L)])
                    return 0
                lax.fori_loop(0, L, wrow, 0)
                pltpu.sync_copy(wrows.at[pl.ds(0, L)],
                                out_hbm.at[pl.ds(lo + myrow + blk * L, L)])
                return 0
            lax.fori_loop(0, SH // L, wblk, 0)
            return 0

        lax.fori_loop(0, NCHUNK // NC, chunk_body, 0)

    return k(xp, src, dst, w0, w1, bias)


def kernel(x, undirected_edges_small_middle, W, att_src, att_dst, bias):
    N = x.shape[0]
    H, C = att_src.shape
    HC = H * C

    A = jnp.zeros((HC, 128), jnp.float32)
    A = A.at[0:C, 0].set(att_src[0]).at[C:HC, 1].set(att_src[1])
    A = A.at[0:C, 2].set(att_dst[0]).at[C:HC, 3].set(att_dst[1])

    xp, a_all = _tc_project(x, W, A)
    a4 = a_all[:, :4].reshape(-1)

    ei = undirected_edges_small_middle.astype(jnp.int32)
    E = ei.shape[1]
    loops = jnp.arange(N, dtype=jnp.int32)
    e_tot = E + N
    e_pad = -(-e_tot // (NC * NS * L)) * (NC * NS * L)
    npad = e_pad - e_tot
    pad_src = (jnp.arange(npad, dtype=jnp.int32) * 613) % N
    pad_dst = jnp.full((npad,), jnp.int32(1 << 30))
    src = jnp.concatenate([ei[0], loops, pad_src])
    dst = jnp.concatenate([ei[1], loops, pad_dst])

    w0, w1 = _sc_weights(src, dst, a4, N)
    out_pad = _sc_scatter(xp, src, dst, w0, w1, bias)
    return out_pad[:N]


# speedup vs baseline: 1.7531x; 1.0021x over previous
import functools

import jax
import jax.numpy as jnp
from jax import lax
from jax.experimental import pallas as pl
from jax.experimental.pallas import tpu as pltpu
from jax.experimental.pallas import tpu_sc as plsc

NC = 2
NS = 16
L = 16

R = 1280
NCHUNK = 8
B = 64


def _tc_project(x, W, A):
    N, IN = x.shape
    HC = W.shape[1]
    BLK = 2000 if N % 2000 == 0 else N

    def body(x_ref, w_ref, a_ref, xp_ref, aall_ref):
        xp = jnp.dot(x_ref[...], w_ref[...], preferred_element_type=jnp.float32)
        xp_ref[...] = xp
        aall_ref[...] = jnp.dot(xp, a_ref[...], preferred_element_type=jnp.float32)

    return pl.pallas_call(
        body,
        grid=(N // BLK,),
        in_specs=[
            pl.BlockSpec((BLK, IN), lambda i: (i, 0)),
            pl.BlockSpec((IN, HC), lambda i: (0, 0)),
            pl.BlockSpec((HC, 128), lambda i: (0, 0)),
        ],
        out_specs=[
            pl.BlockSpec((BLK, HC), lambda i: (i, 0)),
            pl.BlockSpec((BLK, 128), lambda i: (i, 0)),
        ],
        out_shape=[
            jax.ShapeDtypeStruct((N, HC), jnp.float32),
            jax.ShapeDtypeStruct((N, 128), jnp.float32),
        ],
    )(x, W, A)


def _sc_weights(src, dst, a4, n_nodes):
    E_pad = src.shape[0]
    SPAN = E_pad // (NC * NS)
    N = n_nodes

    mesh = plsc.VectorSubcoreMesh(
        core_axis_name="c", subcore_axis_name="s", num_cores=NC, num_subcores=NS)

    @functools.partial(
        pl.kernel,
        mesh=mesh,
        compiler_params=pltpu.CompilerParams(
            use_tc_tiling_on_sc=False, needs_layout_passes=False),
        out_type=[
            jax.ShapeDtypeStruct((E_pad,), jnp.float32),
            jax.ShapeDtypeStruct((E_pad,), jnp.float32),
        ],
        scratch_types=[
            pltpu.VMEM((N * 4,), jnp.float32),
            pltpu.VMEM((SPAN,), jnp.int32),
            pltpu.VMEM((SPAN,), jnp.int32),
            pltpu.VMEM((SPAN,), jnp.float32),
            pltpu.VMEM((SPAN,), jnp.float32),
        ],
    )
    def k(src_hbm, dst_hbm, a4_hbm, w0_hbm, w1_hbm,
          a4_v, src_v, dst_v, w0_v, w1_v):
        cid = lax.axis_index("c")
        sid = lax.axis_index("s")
        base = (cid * NS + sid) * SPAN
        pltpu.sync_copy(a4_hbm, a4_v)
        pltpu.sync_copy(src_hbm.at[pl.ds(base, SPAN)], src_v)
        pltpu.sync_copy(dst_hbm.at[pl.ds(base, SPAN)], dst_v)

        def grp(g, _):
            s16 = src_v[pl.ds(g * L, L)]
            d16 = dst_v[pl.ds(g * L, L)]
            valid = d16 < jnp.int32(N)
            s4 = s16 * 4
            d4 = d16 * 4
            as0 = plsc.load_gather(a4_v, [s4], mask=valid)
            as1 = plsc.load_gather(a4_v, [s4 + 1], mask=valid)
            ad0 = plsc.load_gather(a4_v, [d4 + 2], mask=valid)
            ad1 = plsc.load_gather(a4_v, [d4 + 3], mask=valid)
            al0 = as0 + ad0
            al1 = as1 + ad1
            al0 = jnp.where(al0 > 0, al0, 0.2 * al0)
            al1 = jnp.where(al1 > 0, al1, 0.2 * al1)
            w0_v[pl.ds(g * L, L)] = jnp.where(valid, jnp.exp(al0), 0.0)
            w1_v[pl.ds(g * L, L)] = jnp.where(valid, jnp.exp(al1), 0.0)
            return 0

        lax.fori_loop(0, SPAN // L, grp, 0)
        pltpu.sync_copy(w0_v, w0_hbm.at[pl.ds(base, SPAN)])
        pltpu.sync_copy(w1_v, w1_hbm.at[pl.ds(base, SPAN)])

    return k(src, dst, a4)


def _sc_scatter(xp, src, dst, w0, w1, bias):
    N, HC = xp.shape
    KV = HC // L
    E_pad = src.shape[0]
    SPAN = E_pad // NS
    GROUPS = SPAN // L
    EBLK_G = 111
    EBLK = EBLK_G * L
    NBLK = GROUPS // EBLK_G
    NPAD = R * NCHUNK
    SH = R // NS

    assert GROUPS % EBLK_G == 0

    mesh = plsc.VectorSubcoreMesh(
        core_axis_name="c", subcore_axis_name="s", num_cores=NC, num_subcores=NS)

    buf_types = []
    for _ in range(2):
        buf_types += [
            pltpu.VMEM((B, HC), jnp.float32),
            pltpu.VMEM((B,), jnp.int32),
            pltpu.VMEM((B,), jnp.int32),
            pltpu.VMEM((B,), jnp.float32),
            pltpu.VMEM((B,), jnp.float32),
            pltpu.VMEM((B,), jnp.int32),
            pltpu.SemaphoreType.DMA,
            pltpu.SemaphoreType.DMA,
        ]

    @functools.partial(
        pl.kernel,
        mesh=mesh,
        compiler_params=pltpu.CompilerParams(
            use_tc_tiling_on_sc=False, needs_layout_passes=False),
        out_type=jax.ShapeDtypeStruct((NPAD, HC), jnp.float32),
        scratch_types=[
            pltpu.VMEM_SHARED((R, HC), jnp.float32),
            pltpu.VMEM_SHARED((NS, R), jnp.float32),
            pltpu.VMEM_SHARED((NS, R), jnp.float32),
            pltpu.VMEM((EBLK,), jnp.int32),
            pltpu.VMEM((EBLK,), jnp.int32),
            pltpu.VMEM((EBLK,), jnp.float32),
            pltpu.VMEM((EBLK,), jnp.float32),
            pltpu.VMEM((R,), jnp.float32),
            pltpu.VMEM((R,), jnp.float32),
            pltpu.VMEM((HC,), jnp.float32),
            pltpu.VMEM((NS, SH), jnp.float32),
            pltpu.VMEM((NS, SH), jnp.float32),
            pltpu.VMEM((L,), jnp.float32),
            pltpu.VMEM((L,), jnp.float32),
            pltpu.VMEM((B,), jnp.int32),
            pltpu.VMEM((B,), jnp.int32),
            pltpu.VMEM((B,), jnp.float32),
            pltpu.VMEM((B,), jnp.float32),
        ] + buf_types,
    )
    def k(xp_hbm, src_hbm, dst_hbm, w0_hbm, w1_hbm, bias_hbm, out_hbm,
          slab, dall0, dall1, eb_src, eb_dst, eb_w0, eb_w1,
          den0, den1, bias_v, dc0, dc1, inv0_v, inv1_v,
          st_src, st_loc, st_w0, st_w1,
          rows_a, wls_a, wll_a, ww0_a, ww1_a, sloc_a, gsem_a, ssem_a,
          rows_b, wls_b, wll_b, ww0_b, ww1_b, sloc_b, gsem_b, ssem_b):
        cid = lax.axis_index("c")
        sid = lax.axis_index("s")
        base = sid * SPAN
        pltpu.sync_copy(bias_hbm, bias_v)

        bufs = ((rows_a, wls_a, wll_a, ww0_a, ww1_a, sloc_a, gsem_a, ssem_a),
                (rows_b, wls_b, wll_b, ww0_b, ww1_b, sloc_b, gsem_b, ssem_b))

        zf = jnp.zeros((L,), jnp.float32)
        iot = lax.iota(jnp.int32, L)

        def resentinel():
            for b in range(B // L):
                st_src[pl.ds(b * L, L)] = (iot * 601 + b * 977) % N
                st_loc[pl.ds(b * L, L)] = (iot * 37 + b * 53) % R
                st_w0[pl.ds(b * L, L)] = zf
                st_w1[pl.ds(b * L, L)] = zf

        def snapshot(p):
            _, wls, wll, ww0, ww1, _, _, _ = bufs[p]
            for b in range(B // L):
                sl = pl.ds(b * L, L)
                wls[sl] = st_src[sl]
                wll[sl] = st_loc[sl]
                ww0[sl] = st_w0[sl]
                ww1[sl] = st_w1[sl]
            resentinel()

        def zero_rowsf():
            rowz = bufs[0][0]

            def zrow(e, _):
                row = rowz.at[e]
                for kk in range(KV):
                    row[pl.ds(kk * L, L)] = zf
                return 0
            lax.fori_loop(0, B, zrow, 0)

        def start_gather(p):
            rows, wls = bufs[p][0], bufs[p][1]
            pltpu.make_async_copy(xp_hbm.at[wls], rows, bufs[p][6]).start()

        def wait_scatter(p):
            rows, sloc, ssem = bufs[p][0], bufs[p][5], bufs[p][7]
            pltpu.make_async_copy(rows, slab.at[sloc], ssem).wait()

        def drain(p):
            rows, wls, wll, ww0, ww1, sloc, gsem, ssem = bufs[p]
            pltpu.make_async_copy(xp_hbm.at[wls], rows, gsem).wait()
            for b in range(B // L):
                sloc[pl.ds(b * L, L)] = wll[pl.ds(b * L, L)]

            def scale(e, _):
                ev = jnp.full((L,), e, jnp.int32)
                wv0 = plsc.load_gather(ww0, [ev])
                wv1 = plsc.load_gather(ww1, [ev])
                row = rows.at[e]
                for kk in range(KV // 2):
                    row[pl.ds(kk * L, L)] = row[pl.ds(kk * L, L)] * wv0
                for kk in range(KV // 2, KV):
                    row[pl.ds(kk * L, L)] = row[pl.ds(kk * L, L)] * wv1
                return 0
            lax.fori_loop(0, B, scale, 0)
            pltpu.make_async_copy(rows, slab.at[sloc], ssem).start(add=True)

        resentinel()
        myrow = sid * SH

        def chunk_body(ci, _):
            chunk = NC * ci + cid
            lo = chunk * R

            zero_rowsf()
            rowz = bufs[0][0]
            pltpu.sync_copy(rowz, slab.at[pl.ds(myrow, B)])
            pltpu.sync_copy(rowz.at[pl.ds(0, SH - B)],
                            slab.at[pl.ds(myrow + B, SH - B)])

            def zden(i, _):
                den0[pl.ds(i * L, L)] = zf
                den1[pl.ds(i * L, L)] = zf
                return 0
            lax.fori_loop(0, R // L, zden, 0)
            plsc.subcore_barrier()

            def block(bi, carry):
                eoff = base + bi * EBLK
                pltpu.sync_copy(src_hbm.at[pl.ds(eoff, EBLK)], eb_src)
                pltpu.sync_copy(dst_hbm.at[pl.ds(eoff, EBLK)], eb_dst)
                pltpu.sync_copy(w0_hbm.at[pl.ds(eoff, EBLK)], eb_w0)
                pltpu.sync_copy(w1_hbm.at[pl.ds(eoff, EBLK)], eb_w1)

                def group(g, carry):
                    cnt, fb, warm, sp0, sp1 = carry
                    sp = (sp0, sp1)
                    s16 = eb_src[pl.ds(g * L, L)]
                    d16 = eb_dst[pl.ds(g * L, L)]
                    w016 = eb_w0[pl.ds(g * L, L)]
                    w116 = eb_w1[pl.ds(g * L, L)]
                    m = (d16 >= lo) & (d16 < lo + R)
                    loc16 = d16 - lo
                    plsc.addupdate_scatter(den0, [loc16], w016, mask=m)
                    plsc.addupdate_scatter(den1, [loc16], w116, mask=m)
                    plsc.store_compressed(st_src.at[pl.ds(cnt, L)], s16, mask=m)
                    plsc.store_compressed(st_loc.at[pl.ds(cnt, L)], loc16, mask=m)
                    plsc.store_compressed(st_w0.at[pl.ds(cnt, L)], w016, mask=m)
                    plsc.store_compressed(st_w1.at[pl.ds(cnt, L)], w116, mask=m)
                    cnt = cnt + plsc.all_reduce_population_count(m)[0]
                    full = cnt > B - L
                    for p in range(2):
                        @pl.when(full & (fb == p))
                        def _():
                            @pl.when(sp[p] == 1)
                            def _():
                                wait_scatter(p)
                            snapshot(p)
                            start_gather(p)

                            @pl.when(warm == 1)
                            def _():
                                drain(1 - p)
                    sp0 = jnp.where(full, jnp.where(fb == 0, 0, warm), sp0)
                    sp1 = jnp.where(full, jnp.where(fb == 1, 0, warm), sp1)
                    cnt = jnp.where(full, jnp.int32(0), cnt)
                    warm = jnp.where(full, jnp.int32(1), warm)
                    fb = jnp.where(full, 1 - fb, fb)
                    return (cnt, fb, warm, sp0, sp1)

                return lax.fori_loop(0, EBLK_G, group, carry)

            cnt, fb, warm, sp0, sp1 = lax.fori_loop(
                0, NBLK, block,
                (jnp.int32(0), jnp.int32(0), jnp.int32(0),
                 jnp.int32(0), jnp.int32(0)))
            sp = (sp0, sp1)

            for p in range(2):
                @pl.when((warm == 1) & (fb == p))
                def _():
                    drain(1 - p)
            for p in range(2):
                @pl.when(fb == p)
                def _():
                    @pl.when(sp[p] == 1)
                    def _():
                        wait_scatter(p)
                    snapshot(p)
                    start_gather(p)
                    drain(p)
                    wait_scatter(p)
            for p in range(2):
                @pl.when((warm == 1) & (fb == p))
                def _():
                    wait_scatter(1 - p)
            plsc.subcore_barrier()

            pltpu.sync_copy(den0, dall0.at[sid])
            pltpu.sync_copy(den1, dall1.at[sid])
            plsc.subcore_barrier()

            def dfetch(t, _):
                pltpu.sync_copy(dall0.at[t, pl.ds(myrow, SH)], dc0.at[t])
                pltpu.sync_copy(dall1.at[t, pl.ds(myrow, SH)], dc1.at[t])
                return 0
            lax.fori_loop(0, NS, dfetch, 0)

            def dsum(t, _):
                for q in range(SH // L):
                    dc0[0, pl.ds(q * L, L)] = (dc0[0, pl.ds(q * L, L)]
                                               + dc0[t, pl.ds(q * L, L)])
                    dc1[0, pl.ds(q * L, L)] = (dc1[0, pl.ds(q * L, L)]
                                               + dc1[t, pl.ds(q * L, L)])
                return 0
            lax.fori_loop(1, NS, dsum, 0)

            def wblk(blk, _):
                wrows = bufs[0][0]
                pltpu.sync_copy(slab.at[pl.ds(myrow + blk * L, L)],
                                wrows.at[pl.ds(0, L)])
                inv0_v[...] = 1.0 / jnp.maximum(dc0[0, pl.ds(blk * L, L)], 1e-16)
                inv1_v[...] = 1.0 / jnp.maximum(dc1[0, pl.ds(blk * L, L)], 1e-16)

                def wrow(e, _):
                    ev = jnp.full((L,), e, jnp.int32)
                    i0 = plsc.load_gather(inv0_v, [ev])
                    i1 = plsc.load_gather(inv1_v, [ev])
                    row = wrows.at[e]
                    for kk in range(KV // 2):
                        row[pl.ds(kk * L, L)] = (row[pl.ds(kk * L, L)] * i0
                                                 + bias_v[pl.ds(kk * L, L)])
                    for kk in range(KV // 2, KV):
                        row[pl.ds(kk * L, L)] = (row[pl.ds(kk * L, L)] * i1
                                                 + bias_v[pl.ds(kk * L, L)])
                    return 0
                lax.fori_loop(0, L, wrow, 0)
                pltpu.sync_copy(wrows.at[pl.ds(0, L)],
                                out_hbm.at[pl.ds(lo + myrow + blk * L, L)])
                return 0
            lax.fori_loop(0, SH // L, wblk, 0)
            return 0

        lax.fori_loop(0, NCHUNK // NC, chunk_body, 0)

    return k(xp, src, dst, w0, w1, bias)


def kernel(x, undirected_edges_small_middle, W, att_src, att_dst, bias):
    N = x.shape[0]
    H, C = att_src.shape
    HC = H * C

    A = jnp.zeros((HC, 128), jnp.float32)
    A = A.at[0:C, 0].set(att_src[0]).at[C:HC, 1].set(att_src[1])
    A = A.at[0:C, 2].set(att_dst[0]).at[C:HC, 3].set(att_dst[1])

    xp, a_all = _tc_project(x, W, A)
    a4 = a_all[:, :4].reshape(-1)

    ei = undirected_edges_small_middle.astype(jnp.int32)
    E = ei.shape[1]
    loops = jnp.arange(N, dtype=jnp.int32)
    e_tot = E + N
    e_pad = -(-e_tot // (NC * NS * L)) * (NC * NS * L)
    npad = e_pad - e_tot
    pad_src = (jnp.arange(npad, dtype=jnp.int32) * 613) % N
    pad_dst = jnp.full((npad,), jnp.int32(1 << 30))
    src = jnp.concatenate([ei[0], loops, pad_src])
    dst = jnp.concatenate([ei[1], loops, pad_dst])

    w0, w1 = _sc_weights(src, dst, a4, N)
    out_pad = _sc_scatter(xp, src, dst, w0, w1, bias)
    return out_pad[:N]
